# Initial kernel scaffold; baseline (speedup 1.0000x reference)
import jax
import jax.numpy as jnp
from jax.experimental import pallas as pl


def kernel(input_field, height_map_sqrt):
    raise NotImplementedError("write your pallas kernel here")


# speedup vs baseline: 151.7040x; 151.7040x over previous
import functools

import numpy as np
import jax
import jax.numpy as jnp
from jax import lax
from jax.experimental import pallas as pl
from jax.experimental.pallas import tpu as pltpu
from jax.experimental.pallas import tpu_sc as plsc

H = 2048
N2 = 2 * H
MAX_HEIGHT = np.float32(1.2e-06)
PHI_SCALE = np.float32((2.0 * np.pi / 5.5e-07) * (1.5 - 1.0))

NC = 2
NS = 16
L = 16
NW = NC * NS
ROWS_PER_W = N2 // NW
VECS_PER_ROW = N2 // L


def _isqrt_exact(n):
    nf = n.astype(jnp.float32)
    gi = jnp.int32(0x5F3759DF) - lax.shift_right_arithmetic(
        lax.bitcast_convert_type(nf, jnp.int32), 1)
    g = lax.bitcast_convert_type(gi, jnp.float32)
    hn = nf * jnp.float32(0.5)
    g = g * (jnp.float32(1.5) - hn * g * g)
    g = g * (jnp.float32(1.5) - hn * g * g)
    r = (nf * g).astype(jnp.int32)
    rp = r + 1
    r = jnp.where(rp * rp <= n, rp, r)
    r = jnp.where(r * r > n, r - 1, r)
    return r


def _sc_radial_map(hms_flat):
    mesh = plsc.VectorSubcoreMesh(core_axis_name="c", subcore_axis_name="s")

    @functools.partial(
        pl.kernel,
        out_type=jax.ShapeDtypeStruct((N2, N2), jnp.float32),
        mesh=mesh,
        compiler_params=pltpu.CompilerParams(needs_layout_passes=False),
        scratch_types=[
            pltpu.VMEM((H,), jnp.float32),
            pltpu.VMEM((N2,), jnp.float32),
            pltpu.VMEM((N2,), jnp.int32),
            pltpu.VMEM((2 * N2,), jnp.float32),
            pltpu.SemaphoreType.DMA,
        ],
    )
    def sc_kernel(hms_hbm, out_hbm, hms_v, table_v, ysq_v, rowbuf_v, sem):
        cid = lax.axis_index("c")
        sid = lax.axis_index("s")
        wid = sid * NC + cid

        pltpu.sync_copy(hms_hbm, hms_v)
        iota = lax.iota(jnp.int32, L)

        def ysq_init(i, _):
            base = i * L
            y = base + iota - H
            ysq_v[pl.ds(base, L)] = y * y
            return 0

        lax.fori_loop(0, VECS_PER_ROW, ysq_init, 0)

        def table_init(i, _):
            base = i * L
            x = hms_v[pl.ds(base, L)]
            table_v[pl.ds(base, L)] = lax.rem(x * x, MAX_HEIGHT)
            table_v[pl.ds(H + base, L)] = jnp.zeros((L,), jnp.float32)
            return 0

        lax.fori_loop(0, H // L, table_init, 0)

        row0 = wid * ROWS_PER_W

        def row_pair(pp, _):
            for b in range(2):
                rr = pp * 2 + b
                row = row0 + rr
                x = row - H
                xsq = x * x
                bbase = b * N2

                @pl.when(pp >= 1)
                def _wait_prev():
                    pltpu.make_async_copy(
                        rowbuf_v.at[pl.ds(bbase, N2)], out_hbm.at[row], sem
                    ).wait()

                def col_body(j, _):
                    cbase = j * L
                    n = xsq + ysq_v[pl.ds(cbase, L)]
                    r = _isqrt_exact(n)
                    rowbuf_v[pl.ds(bbase + cbase, L)] = plsc.load_gather(
                        table_v, [r])
                    return 0

                lax.fori_loop(0, VECS_PER_ROW, col_body, 0, unroll=4)
                pltpu.async_copy(
                    rowbuf_v.at[pl.ds(bbase, N2)], out_hbm.at[row], sem)
            return 0

        lax.fori_loop(0, ROWS_PER_W // 2, row_pair, 0)
        pltpu.make_async_copy(
            rowbuf_v.at[pl.ds(0, N2)], out_hbm.at[row0], sem).wait()
        pltpu.make_async_copy(
            rowbuf_v.at[pl.ds(N2, N2)], out_hbm.at[row0], sem).wait()

    return sc_kernel(hms_flat)


def _tc_body(x_ref, h_ref, hm_ref, re_ref, im_ref):
    h = h_ref[...]
    hm = lax.rem(h * h, MAX_HEIGHT)
    hm_ref[...] = hm
    phi = hm * PHI_SCALE
    c = jnp.cos(phi)
    s = jnp.sin(phi)
    x = x_ref[...]
    re_ref[...] = x * c
    im_ref[...] = x * s


def _tc_small(x3, h2):
    return pl.pallas_call(
        _tc_body,
        out_shape=[
            jax.ShapeDtypeStruct((16, 128), jnp.float32),
            jax.ShapeDtypeStruct((4, 16, 128), jnp.float32),
            jax.ShapeDtypeStruct((4, 16, 128), jnp.float32),
        ],
    )(x3, h2)


def kernel(input_field, height_map_sqrt):
    x3 = input_field.reshape(4, 16, 128)
    h2 = height_map_sqrt.reshape(16, 128)
    hm2, re3, im3 = _tc_small(x3, h2)
    hma = _sc_radial_map(height_map_sqrt.reshape(H))
    out = lax.complex(re3.reshape(4, H, 1, 1), im3.reshape(4, H, 1, 1))
    height_map = hm2.reshape(1, H, 1, 1)
    height_map_all = hma.reshape(1, N2, N2, 1)
    return out, height_map, height_map_all


# speedup vs baseline: 709.5618x; 4.6773x over previous
import functools

import numpy as np
import jax
import jax.numpy as jnp
from jax import lax
from jax.experimental import pallas as pl
from jax.experimental.pallas import tpu as pltpu
from jax.experimental.pallas import tpu_sc as plsc

H = 2048
N2 = 2 * H
MAX_HEIGHT = np.float32(1.2e-06)
PHI_SCALE = np.float32((2.0 * np.pi / 5.5e-07) * (1.5 - 1.0))

NC = 2
NS = 16
L = 16
NW = NC * NS
ROWS_PER_W = N2 // NW
VECS_PER_ROW = N2 // L


def _isqrt_exact(n):
    nf = n.astype(jnp.float32)
    gi = jnp.int32(0x5F3759DF) - lax.shift_right_arithmetic(
        lax.bitcast_convert_type(nf, jnp.int32), 1)
    g = lax.bitcast_convert_type(gi, jnp.float32)
    hn = nf * jnp.float32(0.5)
    g = g * (jnp.float32(1.5) - hn * g * g)
    g = g * (jnp.float32(1.5) - hn * g * g)
    r = (nf * g).astype(jnp.int32)
    rp = r + 1
    r = jnp.where(rp * rp <= n, rp, r)
    r = jnp.where(r * r > n, r - 1, r)
    return r


def _sc_radial_map(hms_flat):
    mesh = plsc.VectorSubcoreMesh(core_axis_name="c", subcore_axis_name="s")

    @functools.partial(
        pl.kernel,
        out_type=jax.ShapeDtypeStruct((N2, N2), jnp.float32),
        mesh=mesh,
        compiler_params=pltpu.CompilerParams(needs_layout_passes=False),
        scratch_types=[
            pltpu.VMEM((H,), jnp.float32),
            pltpu.VMEM((N2,), jnp.float32),
            pltpu.VMEM((N2,), jnp.int32),
            pltpu.VMEM((2 * N2,), jnp.float32),
            pltpu.SemaphoreType.DMA,
        ],
    )
    def sc_kernel(hms_hbm, out_hbm, hms_v, table_v, ysq_v, rowbuf_v, sem):
        cid = lax.axis_index("c")
        sid = lax.axis_index("s")
        wid = sid * NC + cid

        pltpu.sync_copy(hms_hbm, hms_v)
        iota = lax.iota(jnp.int32, L)

        def ysq_init(i, _):
            base = i * L
            y = base + iota - H
            ysq_v[pl.ds(base, L)] = y * y
            return 0

        lax.fori_loop(0, VECS_PER_ROW, ysq_init, 0)

        def table_init(i, _):
            base = i * L
            x = hms_v[pl.ds(base, L)]
            table_v[pl.ds(base, L)] = lax.rem(x * x, MAX_HEIGHT)
            table_v[pl.ds(H + base, L)] = jnp.zeros((L,), jnp.float32)
            return 0

        lax.fori_loop(0, H // L, table_init, 0)

        row0 = wid * ROWS_PER_W

        def row_pair(pp, _):
            for b in range(2):
                rr = pp * 2 + b
                row = row0 + rr
                x = row - H
                xsq = x * x
                bbase = b * N2

                @pl.when(pp >= 1)
                def _wait_prev():
                    pltpu.make_async_copy(
                        rowbuf_v.at[pl.ds(bbase, N2)], out_hbm.at[row], sem
                    ).wait()

                @plsc.parallel_loop(0, N2, L, unroll=8)
                def col_body(cbase):
                    n = xsq + ysq_v[pl.ds(cbase, L)]
                    r = _isqrt_exact(n)
                    rowbuf_v[pl.ds(bbase + cbase, L)] = plsc.load_gather(
                        table_v, [r])
                pltpu.async_copy(
                    rowbuf_v.at[pl.ds(bbase, N2)], out_hbm.at[row], sem)
            return 0

        lax.fori_loop(0, ROWS_PER_W // 2, row_pair, 0)
        pltpu.make_async_copy(
            rowbuf_v.at[pl.ds(0, N2)], out_hbm.at[row0], sem).wait()
        pltpu.make_async_copy(
            rowbuf_v.at[pl.ds(N2, N2)], out_hbm.at[row0], sem).wait()

    return sc_kernel(hms_flat)


def _tc_body(x_ref, h_ref, hm_ref, re_ref, im_ref):
    h = h_ref[...]
    hm = lax.rem(h * h, MAX_HEIGHT)
    hm_ref[...] = hm
    phi = hm * PHI_SCALE
    c = jnp.cos(phi)
    s = jnp.sin(phi)
    x = x_ref[...]
    re_ref[...] = x * c
    im_ref[...] = x * s


def _tc_small(x3, h2):
    return pl.pallas_call(
        _tc_body,
        out_shape=[
            jax.ShapeDtypeStruct((16, 128), jnp.float32),
            jax.ShapeDtypeStruct((4, 16, 128), jnp.float32),
            jax.ShapeDtypeStruct((4, 16, 128), jnp.float32),
        ],
    )(x3, h2)


def kernel(input_field, height_map_sqrt):
    x3 = input_field.reshape(4, 16, 128)
    h2 = height_map_sqrt.reshape(16, 128)
    hm2, re3, im3 = _tc_small(x3, h2)
    hma = _sc_radial_map(height_map_sqrt.reshape(H))
    out = lax.complex(re3.reshape(4, H, 1, 1), im3.reshape(4, H, 1, 1))
    height_map = hm2.reshape(1, H, 1, 1)
    height_map_all = hma.reshape(1, N2, N2, 1)
    return out, height_map, height_map_all


# speedup vs baseline: 1342.2414x; 1.8916x over previous
import functools

import numpy as np
import jax
import jax.numpy as jnp
from jax import lax
from jax.experimental import pallas as pl
from jax.experimental.pallas import tpu as pltpu
from jax.experimental.pallas import tpu_sc as plsc

H = 2048
N2 = 2 * H
MAX_HEIGHT = np.float32(1.2e-06)
PHI_SCALE = np.float32((2.0 * np.pi / 5.5e-07) * (1.5 - 1.0))

NC = 2
NS = 16
L = 16
NW = NC * NS


def _isqrt_exact(n):
    nf = n.astype(jnp.float32)
    gi = jnp.int32(0x5F3759DF) - lax.shift_right_arithmetic(
        lax.bitcast_convert_type(nf, jnp.int32), 1)
    g = lax.bitcast_convert_type(gi, jnp.float32)
    hn = nf * jnp.float32(0.5)
    g = g * (jnp.float32(1.5) - hn * g * g)
    g = g * (jnp.float32(1.5) - hn * g * g)
    r = (nf * g).astype(jnp.int32)
    rp = r + 1
    r = jnp.where(rp * rp <= n, rp, r)
    r = jnp.where(r * r > n, r - 1, r)
    return r


def _sc_radial_map(hms_flat):
    mesh = plsc.VectorSubcoreMesh(core_axis_name="c", subcore_axis_name="s")
    rows_per_w = H // NW

    @functools.partial(
        pl.kernel,
        out_type=jax.ShapeDtypeStruct((N2, N2), jnp.float32),
        mesh=mesh,
        compiler_params=pltpu.CompilerParams(needs_layout_passes=False),
        scratch_types=[
            pltpu.VMEM((H,), jnp.float32),
            pltpu.VMEM((N2,), jnp.float32),
            pltpu.VMEM((H + L,), jnp.int32),
            pltpu.VMEM((H + L,), jnp.float32),
            pltpu.VMEM((2 * N2,), jnp.float32),
            pltpu.SemaphoreType.DMA,
        ],
    )
    def sc_kernel(hms_hbm, out_hbm, hms_v, table_v, dsq_v, rhalf_v,
                  rowbuf_v, sem):
        cid = lax.axis_index("c")
        sid = lax.axis_index("s")
        wid = sid * NC + cid

        pltpu.sync_copy(hms_hbm, hms_v)
        iota = lax.iota(jnp.int32, L)

        @plsc.parallel_loop(0, H + L, L, unroll=4)
        def dsq_init(base):
            d = base + iota
            dsq_v[pl.ds(base, L)] = d * d

        @plsc.parallel_loop(0, H, L, unroll=4)
        def table_init(base):
            x = hms_v[pl.ds(base, L)]
            table_v[pl.ds(base, L)] = lax.rem(x * x, MAX_HEIGHT)
            table_v[pl.ds(H + base, L)] = jnp.zeros((L,), jnp.float32)

        def fill_row(row, bbase):
            x = row - H
            xsq = x * x

            @plsc.parallel_loop(0, H, L, unroll=8)
            def right_body(dbase):
                n = xsq + dsq_v[pl.ds(dbase, L)]
                r = _isqrt_exact(n)
                vals = plsc.load_gather(table_v, [r])
                rowbuf_v[pl.ds(bbase + H + dbase, L)] = vals
                rhalf_v[pl.ds(dbase, L)] = vals

            n_t = xsq + dsq_v[pl.ds(H, L)]
            rhalf_v[pl.ds(H, L)] = plsc.load_gather(
                table_v, [_isqrt_exact(n_t)])

            @plsc.parallel_loop(0, H, L, unroll=8)
            def left_body(jbase):
                v = rhalf_v[pl.ds(H - jbase - (L - 1), L)]
                rowbuf_v[pl.ds(bbase + jbase, L)] = lax.rev(v, (0,))

        row0 = wid * rows_per_w

        def row_pair(pp, _):
            for b in range(2):
                rr = pp * 2 + b
                row = row0 + rr
                bbase = b * N2

                @pl.when(pp >= 1)
                def _wait_prev():
                    pltpu.make_async_copy(
                        rowbuf_v.at[pl.ds(bbase, N2)],
                        out_hbm.at[row], sem).wait()

                    @pl.when(row - 2 >= 1)
                    def _wait_mirror():
                        pltpu.make_async_copy(
                            rowbuf_v.at[pl.ds(bbase, N2)],
                            out_hbm.at[row], sem).wait()

                fill_row(row, bbase)
                pltpu.async_copy(
                    rowbuf_v.at[pl.ds(bbase, N2)],
                    out_hbm.at[row], sem)

                @pl.when(row >= 1)
                def _mirror():
                    pltpu.async_copy(
                        rowbuf_v.at[pl.ds(bbase, N2)],
                        out_hbm.at[N2 - row], sem)
            return 0

        lax.fori_loop(0, rows_per_w // 2, row_pair, 0)

        for _ in range(2):
            for b in range(2):
                pltpu.make_async_copy(
                    rowbuf_v.at[pl.ds(b * N2, N2)],
                    out_hbm.at[0], sem).wait()

        @pl.when(wid == NW - 1)
        def _center():
            fill_row(H, 0)
            pltpu.async_copy(
                rowbuf_v.at[pl.ds(0, N2)], out_hbm.at[H], sem)
            pltpu.make_async_copy(
                rowbuf_v.at[pl.ds(0, N2)], out_hbm.at[H], sem).wait()

    return sc_kernel(hms_flat)


def _tc_body(x_ref, h_ref, hm_ref, re_ref, im_ref):
    h = h_ref[...]
    hm = lax.rem(h * h, MAX_HEIGHT)
    hm_ref[...] = hm
    phi = hm * PHI_SCALE
    c = jnp.cos(phi)
    s = jnp.sin(phi)
    x = x_ref[...]
    re_ref[...] = x * c
    im_ref[...] = x * s


def _tc_small(x3, h2):
    return pl.pallas_call(
        _tc_body,
        out_shape=[
            jax.ShapeDtypeStruct((16, 128), jnp.float32),
            jax.ShapeDtypeStruct((4, 16, 128), jnp.float32),
            jax.ShapeDtypeStruct((4, 16, 128), jnp.float32),
        ],
    )(x3, h2)


def kernel(input_field, height_map_sqrt):
    x3 = input_field.reshape(4, 16, 128)
    h2 = height_map_sqrt.reshape(16, 128)
    hm2, re3, im3 = _tc_small(x3, h2)
    hma = _sc_radial_map(height_map_sqrt.reshape(H))
    out = lax.complex(re3.reshape(4, H, 1, 1), im3.reshape(4, H, 1, 1))
    height_map = hm2.reshape(1, H, 1, 1)
    return out, height_map, hma.reshape(1, N2, N2, 1)


# speedup vs baseline: 1342.5373x; 1.0002x over previous
import functools

import numpy as np
import jax
import jax.numpy as jnp
from jax import lax
from jax.experimental import pallas as pl
from jax.experimental.pallas import tpu as pltpu
from jax.experimental.pallas import tpu_sc as plsc

H = 2048
N2 = 2 * H
MAX_HEIGHT = np.float32(1.2e-06)
PHI_SCALE = np.float32((2.0 * np.pi / 5.5e-07) * (1.5 - 1.0))

NC = 2
NS = 16
L = 16
NW = NC * NS


def _isqrt_exact(n):
    nf = n.astype(jnp.float32)
    gi = jnp.int32(0x5F3759DF) - lax.shift_right_arithmetic(
        lax.bitcast_convert_type(nf, jnp.int32), 1)
    g = lax.bitcast_convert_type(gi, jnp.float32)
    hn = nf * jnp.float32(0.5)
    g = g * (jnp.float32(1.5) - hn * g * g)
    g = g * (jnp.float32(1.5) - hn * g * g)
    r = (nf * g).astype(jnp.int32)
    rp = r + 1
    r = jnp.where(rp * rp <= n, rp, r)
    r = jnp.where(r * r > n, r - 1, r)
    return r


def _sc_radial_map(hms_flat):
    mesh = plsc.VectorSubcoreMesh(core_axis_name="c", subcore_axis_name="s")
    rows_per_w = H // NW

    @functools.partial(
        pl.kernel,
        out_type=jax.ShapeDtypeStruct((N2, N2), jnp.float32),
        mesh=mesh,
        compiler_params=pltpu.CompilerParams(
            needs_layout_passes=False, use_tc_tiling_on_sc=True),
        scratch_types=[
            pltpu.VMEM((H,), jnp.float32),
            pltpu.VMEM((N2,), jnp.float32),
            pltpu.VMEM((H + L,), jnp.int32),
            pltpu.VMEM((H + L,), jnp.float32),
            pltpu.VMEM((2 * N2,), jnp.float32),
            pltpu.SemaphoreType.DMA,
        ],
    )
    def sc_kernel(hms_hbm, out_hbm, hms_v, table_v, dsq_v, rhalf_v,
                  rowbuf_v, sem):
        cid = lax.axis_index("c")
        sid = lax.axis_index("s")
        wid = sid * NC + cid

        pltpu.sync_copy(hms_hbm, hms_v)
        iota = lax.iota(jnp.int32, L)

        @plsc.parallel_loop(0, H + L, L, unroll=4)
        def dsq_init(base):
            d = base + iota
            dsq_v[pl.ds(base, L)] = d * d

        @plsc.parallel_loop(0, H, L, unroll=4)
        def table_init(base):
            x = hms_v[pl.ds(base, L)]
            table_v[pl.ds(base, L)] = lax.rem(x * x, MAX_HEIGHT)
            table_v[pl.ds(H + base, L)] = jnp.zeros((L,), jnp.float32)

        def fill_row(row, bbase):
            x = row - H
            xsq = x * x

            @plsc.parallel_loop(0, H, L, unroll=8)
            def right_body(dbase):
                n = xsq + dsq_v[pl.ds(dbase, L)]
                r = _isqrt_exact(n)
                vals = plsc.load_gather(table_v, [r])
                rowbuf_v[pl.ds(bbase + H + dbase, L)] = vals
                rhalf_v[pl.ds(dbase, L)] = vals

            n_t = xsq + dsq_v[pl.ds(H, L)]
            rhalf_v[pl.ds(H, L)] = plsc.load_gather(
                table_v, [_isqrt_exact(n_t)])

            @plsc.parallel_loop(0, H, L, unroll=8)
            def left_body(jbase):
                v = rhalf_v[pl.ds(H - jbase - (L - 1), L)]
                rowbuf_v[pl.ds(bbase + jbase, L)] = lax.rev(v, (0,))

        row0 = wid * rows_per_w

        def row_pair(pp, _):
            for b in range(2):
                rr = pp * 2 + b
                row = row0 + rr
                bbase = b * N2

                @pl.when(pp >= 1)
                def _wait_prev():
                    pltpu.make_async_copy(
                        rowbuf_v.at[pl.ds(bbase, N2)],
                        out_hbm.at[row], sem).wait()

                    @pl.when(row - 2 >= 1)
                    def _wait_mirror():
                        pltpu.make_async_copy(
                            rowbuf_v.at[pl.ds(bbase, N2)],
                            out_hbm.at[row], sem).wait()

                fill_row(row, bbase)
                pltpu.async_copy(
                    rowbuf_v.at[pl.ds(bbase, N2)],
                    out_hbm.at[row], sem)

                @pl.when(row >= 1)
                def _mirror():
                    pltpu.async_copy(
                        rowbuf_v.at[pl.ds(bbase, N2)],
                        out_hbm.at[N2 - row], sem)
            return 0

        lax.fori_loop(0, rows_per_w // 2, row_pair, 0)

        for _ in range(2):
            for b in range(2):
                pltpu.make_async_copy(
                    rowbuf_v.at[pl.ds(b * N2, N2)],
                    out_hbm.at[0], sem).wait()

        @pl.when(wid == NW - 1)
        def _center():
            fill_row(H, 0)
            pltpu.async_copy(
                rowbuf_v.at[pl.ds(0, N2)], out_hbm.at[H], sem)
            pltpu.make_async_copy(
                rowbuf_v.at[pl.ds(0, N2)], out_hbm.at[H], sem).wait()

    return sc_kernel(hms_flat)


def _tc_body(x_ref, h_ref, hm_ref, re_ref, im_ref):
    h = h_ref[...]
    hm = lax.rem(h * h, MAX_HEIGHT)
    hm_ref[...] = hm
    phi = hm * PHI_SCALE
    c = jnp.cos(phi)
    s = jnp.sin(phi)
    x = x_ref[...]
    re_ref[...] = x * c
    im_ref[...] = x * s


def _tc_small(x3, h2):
    return pl.pallas_call(
        _tc_body,
        out_shape=[
            jax.ShapeDtypeStruct((16, 128), jnp.float32),
            jax.ShapeDtypeStruct((4, 16, 128), jnp.float32),
            jax.ShapeDtypeStruct((4, 16, 128), jnp.float32),
        ],
    )(x3, h2)


def kernel(input_field, height_map_sqrt):
    x3 = input_field.reshape(4, 16, 128)
    h2 = height_map_sqrt.reshape(16, 128)
    hm2, re3, im3 = _tc_small(x3, h2)
    hma = _sc_radial_map(height_map_sqrt.reshape(H))
    out = lax.complex(re3.reshape(4, H, 1, 1), im3.reshape(4, H, 1, 1))
    height_map = hm2.reshape(1, H, 1, 1)
    return out, height_map, hma.reshape(1, N2, N2, 1)


# speedup vs baseline: 1680.6823x; 1.2519x over previous
import functools

import numpy as np
import jax
import jax.numpy as jnp
from jax import lax
from jax.experimental import pallas as pl
from jax.experimental.pallas import tpu as pltpu
from jax.experimental.pallas import tpu_sc as plsc

H = 2048
N2 = 2 * H
MAX_HEIGHT = np.float32(1.2e-06)
PHI_SCALE = np.float32((2.0 * np.pi / 5.5e-07) * (1.5 - 1.0))

NC = 2
NS = 16
L = 16
NW = NC * NS


def _isqrt_exact(n):
    nf = n.astype(jnp.float32)
    gi = jnp.int32(0x5F3759DF) - lax.shift_right_arithmetic(
        lax.bitcast_convert_type(nf, jnp.int32), 1)
    g = lax.bitcast_convert_type(gi, jnp.float32)
    hn = nf * jnp.float32(0.5)
    g = g * (jnp.float32(1.5) - hn * g * g)
    g = g * (jnp.float32(1.5) - hn * g * g)
    r = (nf * g).astype(jnp.int32)
    rp = r + 1
    r = jnp.where(rp * rp <= n, rp, r)
    r = jnp.where(r * r > n, r - 1, r)
    return r


def _sc_radial_map(hms_flat):
    mesh = plsc.VectorSubcoreMesh(core_axis_name="c", subcore_axis_name="s")
    rows_per_w = H // NW

    @functools.partial(
        pl.kernel,
        out_type=jax.ShapeDtypeStruct((N2, N2), jnp.float32),
        mesh=mesh,
        compiler_params=pltpu.CompilerParams(needs_layout_passes=False),
        scratch_types=[
            pltpu.VMEM((H,), jnp.float32),
            pltpu.VMEM((N2,), jnp.float32),
            pltpu.VMEM((H + L,), jnp.int32),
            pltpu.VMEM((H + L,), jnp.int32),
            pltpu.VMEM((H + L,), jnp.float32),
            pltpu.VMEM((2 * N2,), jnp.float32),
            pltpu.SemaphoreType.DMA,
        ],
    )
    def sc_kernel(hms_hbm, out_hbm, hms_v, table_v, dsq_v, rcur_v, rhalf_v,
                  rowbuf_v, sem):
        cid = lax.axis_index("c")
        sid = lax.axis_index("s")
        wid = sid * NC + cid

        pltpu.sync_copy(hms_hbm, hms_v)
        iota = lax.iota(jnp.int32, L)

        @plsc.parallel_loop(0, H + L, L, unroll=4)
        def dsq_init(base):
            d = base + iota
            dsq_v[pl.ds(base, L)] = d * d

        @plsc.parallel_loop(0, H, L, unroll=4)
        def table_init(base):
            x = hms_v[pl.ds(base, L)]
            table_v[pl.ds(base, L)] = lax.rem(x * x, MAX_HEIGHT)
            table_v[pl.ds(H + base, L)] = jnp.zeros((L,), jnp.float32)

        def mirror_left(bbase):
            @plsc.parallel_loop(0, H, L, unroll=8)
            def left_body(jbase):
                v = rhalf_v[pl.ds(H - jbase - (L - 1), L)]
                rowbuf_v[pl.ds(bbase + jbase, L)] = lax.rev(v, (0,))

        def fill_row_boot(row, bbase):
            x = row - H
            xsq = x * x

            @plsc.parallel_loop(0, H, L, unroll=8)
            def right_body(dbase):
                n = xsq + dsq_v[pl.ds(dbase, L)]
                r = _isqrt_exact(n)
                rcur_v[pl.ds(dbase, L)] = r
                vals = plsc.load_gather(table_v, [r])
                rowbuf_v[pl.ds(bbase + H + dbase, L)] = vals
                rhalf_v[pl.ds(dbase, L)] = vals

            n_t = xsq + dsq_v[pl.ds(H, L)]
            r_t = _isqrt_exact(n_t)
            rcur_v[pl.ds(H, L)] = r_t
            rhalf_v[pl.ds(H, L)] = plsc.load_gather(table_v, [r_t])

            mirror_left(bbase)

        def fill_row_incr(row, bbase):
            x = row - H
            xsq = x * x

            @plsc.parallel_loop(0, H, L, unroll=8)
            def right_body(dbase):
                n = xsq + dsq_v[pl.ds(dbase, L)]
                r = rcur_v[pl.ds(dbase, L)]
                r = jnp.where(r * r > n, r - 1, r)
                rcur_v[pl.ds(dbase, L)] = r
                vals = plsc.load_gather(table_v, [r])
                rowbuf_v[pl.ds(bbase + H + dbase, L)] = vals
                rhalf_v[pl.ds(dbase, L)] = vals

            n_t = xsq + dsq_v[pl.ds(H, L)]
            r_t = rcur_v[pl.ds(H, L)]
            r_t = jnp.where(r_t * r_t > n_t, r_t - 1, r_t)
            rcur_v[pl.ds(H, L)] = r_t
            rhalf_v[pl.ds(H, L)] = plsc.load_gather(table_v, [r_t])

            mirror_left(bbase)

        row0 = wid * rows_per_w

        def row_pair(pp, _):
            for b in range(2):
                rr = pp * 2 + b
                row = row0 + rr
                bbase = b * N2

                @pl.when(pp >= 1)
                def _wait_prev():
                    pltpu.make_async_copy(
                        rowbuf_v.at[pl.ds(bbase, N2)],
                        out_hbm.at[row], sem).wait()

                    @pl.when(row - 2 >= 1)
                    def _wait_mirror():
                        pltpu.make_async_copy(
                            rowbuf_v.at[pl.ds(bbase, N2)],
                            out_hbm.at[row], sem).wait()

                if b == 0:
                    @pl.when(pp == 0)
                    def _boot():
                        fill_row_boot(row, bbase)

                    @pl.when(pp > 0)
                    def _incr():
                        fill_row_incr(row, bbase)
                else:
                    fill_row_incr(row, bbase)
                pltpu.async_copy(
                    rowbuf_v.at[pl.ds(bbase, N2)],
                    out_hbm.at[row], sem)

                @pl.when(row >= 1)
                def _mirror():
                    pltpu.async_copy(
                        rowbuf_v.at[pl.ds(bbase, N2)],
                        out_hbm.at[N2 - row], sem)
            return 0

        lax.fori_loop(0, rows_per_w // 2, row_pair, 0)

        for _ in range(2):
            for b in range(2):
                pltpu.make_async_copy(
                    rowbuf_v.at[pl.ds(b * N2, N2)],
                    out_hbm.at[0], sem).wait()

        @pl.when(wid == NW - 1)
        def _center():
            fill_row_boot(H, 0)
            pltpu.async_copy(
                rowbuf_v.at[pl.ds(0, N2)], out_hbm.at[H], sem)
            pltpu.make_async_copy(
                rowbuf_v.at[pl.ds(0, N2)], out_hbm.at[H], sem).wait()

    return sc_kernel(hms_flat)


def _tc_body(x_ref, h_ref, hm_ref, re_ref, im_ref):
    h = h_ref[...]
    hm = lax.rem(h * h, MAX_HEIGHT)
    hm_ref[...] = hm
    phi = hm * PHI_SCALE
    c = jnp.cos(phi)
    s = jnp.sin(phi)
    x = x_ref[...]
    re_ref[...] = x * c
    im_ref[...] = x * s


def _tc_small(x3, h2):
    return pl.pallas_call(
        _tc_body,
        out_shape=[
            jax.ShapeDtypeStruct((16, 128), jnp.float32),
            jax.ShapeDtypeStruct((4, 16, 128), jnp.float32),
            jax.ShapeDtypeStruct((4, 16, 128), jnp.float32),
        ],
    )(x3, h2)


def kernel(input_field, height_map_sqrt):
    x3 = input_field.reshape(4, 16, 128)
    h2 = height_map_sqrt.reshape(16, 128)
    hm2, re3, im3 = _tc_small(x3, h2)
    hma = _sc_radial_map(height_map_sqrt.reshape(H))
    out = lax.complex(re3.reshape(4, H, 1, 1), im3.reshape(4, H, 1, 1))
    height_map = hm2.reshape(1, H, 1, 1)
    return out, height_map, hma.reshape(1, N2, N2, 1)


# speedup vs baseline: 1684.2505x; 1.0021x over previous
import functools

import numpy as np
import jax
import jax.numpy as jnp
from jax import lax
from jax.experimental import pallas as pl
from jax.experimental.pallas import tpu as pltpu
from jax.experimental.pallas import tpu_sc as plsc

H = 2048
N2 = 2 * H
MAX_HEIGHT = np.float32(1.2e-06)
PHI_SCALE = np.float32((2.0 * np.pi / 5.5e-07) * (1.5 - 1.0))

NC = 2
NS = 16
L = 16
NW = NC * NS


def _isqrt_exact(n):
    nf = n.astype(jnp.float32)
    gi = jnp.int32(0x5F3759DF) - lax.shift_right_arithmetic(
        lax.bitcast_convert_type(nf, jnp.int32), 1)
    g = lax.bitcast_convert_type(gi, jnp.float32)
    hn = nf * jnp.float32(0.5)
    g = g * (jnp.float32(1.5) - hn * g * g)
    g = g * (jnp.float32(1.5) - hn * g * g)
    r = (nf * g).astype(jnp.int32)
    rp = r + 1
    r = jnp.where(rp * rp <= n, rp, r)
    r = jnp.where(r * r > n, r - 1, r)
    return r


def _sc_radial_map(hms_flat):
    mesh = plsc.VectorSubcoreMesh(core_axis_name="c", subcore_axis_name="s")
    rows_per_w = H // NW

    @functools.partial(
        pl.kernel,
        out_type=jax.ShapeDtypeStruct((N2, N2), jnp.float32),
        mesh=mesh,
        compiler_params=pltpu.CompilerParams(needs_layout_passes=False),
        scratch_types=[
            pltpu.VMEM((H,), jnp.float32),
            pltpu.VMEM((N2,), jnp.float32),
            pltpu.VMEM((H + L,), jnp.int32),
            pltpu.VMEM((H + L,), jnp.int32),
            pltpu.VMEM((H + L,), jnp.float32),
            pltpu.VMEM((4 * N2,), jnp.float32),
            pltpu.SemaphoreType.DMA,
        ],
    )
    def sc_kernel(hms_hbm, out_hbm, hms_v, table_v, dsq_v, rcur_v, rhalf_v,
                  rowbuf_v, sem):
        cid = lax.axis_index("c")
        sid = lax.axis_index("s")
        wid = sid * NC + cid

        pltpu.sync_copy(hms_hbm, hms_v)
        iota = lax.iota(jnp.int32, L)

        @plsc.parallel_loop(0, H + L, L, unroll=4)
        def dsq_init(base):
            d = base + iota
            dsq_v[pl.ds(base, L)] = d * d

        @plsc.parallel_loop(0, H, L, unroll=4)
        def table_init(base):
            x = hms_v[pl.ds(base, L)]
            table_v[pl.ds(base, L)] = lax.rem(x * x, MAX_HEIGHT)
            table_v[pl.ds(H + base, L)] = jnp.zeros((L,), jnp.float32)

        def mirror_left(bbase):
            @plsc.parallel_loop(0, H, L, unroll=8)
            def left_body(jbase):
                v = rhalf_v[pl.ds(H - jbase - (L - 1), L)]
                rowbuf_v[pl.ds(bbase + jbase, L)] = lax.rev(v, (0,))

        def fill_row_boot(row, bbase):
            x = row - H
            xsq = x * x

            @plsc.parallel_loop(0, H, L, unroll=8)
            def right_body(dbase):
                n = xsq + dsq_v[pl.ds(dbase, L)]
                r = _isqrt_exact(n)
                rcur_v[pl.ds(dbase, L)] = r
                vals = plsc.load_gather(table_v, [r])
                rowbuf_v[pl.ds(bbase + H + dbase, L)] = vals
                rhalf_v[pl.ds(dbase, L)] = vals

            n_t = xsq + dsq_v[pl.ds(H, L)]
            r_t = _isqrt_exact(n_t)
            rcur_v[pl.ds(H, L)] = r_t
            rhalf_v[pl.ds(H, L)] = plsc.load_gather(table_v, [r_t])

            mirror_left(bbase)

        def fill_row_incr(row, bbase):
            x = row - H
            xsq = x * x

            @plsc.parallel_loop(0, H, L, unroll=8)
            def right_body(dbase):
                n = xsq + dsq_v[pl.ds(dbase, L)]
                r = rcur_v[pl.ds(dbase, L)]
                r = jnp.where(r * r > n, r - 1, r)
                rcur_v[pl.ds(dbase, L)] = r
                vals = plsc.load_gather(table_v, [r])
                rowbuf_v[pl.ds(bbase + H + dbase, L)] = vals
                rhalf_v[pl.ds(dbase, L)] = vals

            n_t = xsq + dsq_v[pl.ds(H, L)]
            r_t = rcur_v[pl.ds(H, L)]
            r_t = jnp.where(r_t * r_t > n_t, r_t - 1, r_t)
            rcur_v[pl.ds(H, L)] = r_t
            rhalf_v[pl.ds(H, L)] = plsc.load_gather(table_v, [r_t])

            mirror_left(bbase)

        row0 = wid * rows_per_w

        def row_quad(qq, _):
            for b in range(4):
                rr = qq * 4 + b
                row = row0 + rr
                bbase = b * N2

                @pl.when(qq >= 1)
                def _wait_prev():
                    pltpu.make_async_copy(
                        rowbuf_v.at[pl.ds(bbase, N2)],
                        out_hbm.at[row], sem).wait()

                    @pl.when(row - 4 >= 1)
                    def _wait_mirror():
                        pltpu.make_async_copy(
                            rowbuf_v.at[pl.ds(bbase, N2)],
                            out_hbm.at[row], sem).wait()

                if b == 0:
                    @pl.when(qq == 0)
                    def _boot():
                        fill_row_boot(row, bbase)

                    @pl.when(qq > 0)
                    def _incr():
                        fill_row_incr(row, bbase)
                else:
                    fill_row_incr(row, bbase)
                pltpu.async_copy(
                    rowbuf_v.at[pl.ds(bbase, N2)],
                    out_hbm.at[row], sem)

                @pl.when(row >= 1)
                def _mirror():
                    pltpu.async_copy(
                        rowbuf_v.at[pl.ds(bbase, N2)],
                        out_hbm.at[N2 - row], sem)
            return 0

        lax.fori_loop(0, rows_per_w // 4, row_quad, 0)

        for _ in range(2):
            for b in range(4):
                pltpu.make_async_copy(
                    rowbuf_v.at[pl.ds(b * N2, N2)],
                    out_hbm.at[0], sem).wait()

        @pl.when(wid == NW - 1)
        def _center():
            fill_row_boot(H, 0)
            pltpu.async_copy(
                rowbuf_v.at[pl.ds(0, N2)], out_hbm.at[H], sem)
            pltpu.make_async_copy(
                rowbuf_v.at[pl.ds(0, N2)], out_hbm.at[H], sem).wait()

    return sc_kernel(hms_flat)


def _tc_body(x_ref, h_ref, hm_ref, re_ref, im_ref):
    h = h_ref[...]
    hm = lax.rem(h * h, MAX_HEIGHT)
    hm_ref[...] = hm
    phi = hm * PHI_SCALE
    c = jnp.cos(phi)
    s = jnp.sin(phi)
    x = x_ref[...]
    re_ref[...] = x * c
    im_ref[...] = x * s


def _tc_small(x3, h2):
    return pl.pallas_call(
        _tc_body,
        out_shape=[
            jax.ShapeDtypeStruct((16, 128), jnp.float32),
            jax.ShapeDtypeStruct((4, 16, 128), jnp.float32),
            jax.ShapeDtypeStruct((4, 16, 128), jnp.float32),
        ],
    )(x3, h2)


def kernel(input_field, height_map_sqrt):
    x3 = input_field.reshape(4, 16, 128)
    h2 = height_map_sqrt.reshape(16, 128)
    hm2, re3, im3 = _tc_small(x3, h2)
    hma = _sc_radial_map(height_map_sqrt.reshape(H))
    out = lax.complex(re3.reshape(4, H, 1, 1), im3.reshape(4, H, 1, 1))
    height_map = hm2.reshape(1, H, 1, 1)
    return out, height_map, hma.reshape(1, N2, N2, 1)
